# baseline (speedup 1.0000x reference)
import functools

import jax
import jax.numpy as jnp
from jax import lax
from jax.experimental import pallas as pl
from jax.experimental.pallas import tpu as pltpu
from jax.experimental.pallas import tpu_sc as plsc

NC = 2
NS = 16
LN = 16
K = 80


def _pq_body(x_ref, w_ref, cv_ref, o_ref):
    o_ref[...] = (jnp.dot(x_ref[...], w_ref[...],
                          preferred_element_type=jnp.float32) + cv_ref[...])


def _make_pq(B, N, C, H2):
    return pl.pallas_call(
        _pq_body,
        grid=(B,),
        in_specs=[
            pl.BlockSpec((None, N, C), lambda b: (b, 0, 0)),
            pl.BlockSpec((C, H2), lambda b: (0, 0)),
            pl.BlockSpec((None, 1, H2), lambda b: (b, 0, 0)),
        ],
        out_specs=pl.BlockSpec((None, N, H2), lambda b: (b, 0, 0)),
        out_shape=jax.ShapeDtypeStruct((B, N, H2), jnp.float32),
    )


def _sc_body(B, N, E, HH, pq_hbm, ei_hbm, m_hbm,
             psh, qsh, mrsh, mcsh, ibuf, ridx, cidx, pbuf, qbuf, mbuf, zb,
             sg0, sg1, ss0, ss1):
    c = lax.axis_index("c")
    s = lax.axis_index("s")
    rpt = (N // NS) & ~7
    nrem = N - NS * rpt
    eper = E // NS
    nchunks = eper // K
    row0 = s * rpt
    sg = (sg0, sg1)
    ss = (ss0, ss1)
    M = ridx.shape[0]
    ngroups = nchunks // M

    pcol = pl.multiple_of(c * 2 * HH, 8)
    qcol = pl.multiple_of(c * 2 * HH + HH, 8)

    for b in range(B):
        pltpu.sync_copy(pq_hbm.at[b, pl.ds(row0, rpt), pl.ds(pcol, HH)],
                        psh.at[pl.ds(row0, rpt), :])
        pltpu.sync_copy(pq_hbm.at[b, pl.ds(row0, rpt), pl.ds(qcol, HH)],
                        qsh.at[pl.ds(row0, rpt), :])
        zr = zb.shape[0]
        for z in range(rpt // zr):
            pltpu.sync_copy(zb, mrsh.at[pl.ds(row0 + z * zr, zr), :])
            pltpu.sync_copy(zb, mcsh.at[pl.ds(row0 + z * zr, zr), :])

        @pl.when(s == 0)
        def _():
            r0 = NS * rpt
            pltpu.sync_copy(pq_hbm.at[b, pl.ds(r0, nrem), pl.ds(pcol, HH)],
                            psh.at[pl.ds(r0, nrem), :])
            pltpu.sync_copy(pq_hbm.at[b, pl.ds(r0, nrem), pl.ds(qcol, HH)],
                            qsh.at[pl.ds(r0, nrem), :])
            pltpu.sync_copy(zb.at[pl.ds(0, nrem), :],
                            mrsh.at[pl.ds(r0, nrem), :])
            pltpu.sync_copy(zb.at[pl.ds(0, nrem), :],
                            mcsh.at[pl.ds(r0, nrem), :])

        ei0 = (b * E + s * eper) * 2
        plsc.subcore_barrier()

        def fire_gather(j, d):
            pltpu.async_copy(psh.at[ridx.at[j]], pbuf.at[d], sg[d])
            pltpu.async_copy(qsh.at[cidx.at[j]], qbuf.at[d], sg[d])

        def wait_gather(d):
            pltpu.make_async_copy(psh.at[ridx.at[0]], pbuf.at[d],
                                  sg[d]).wait()
            pltpu.make_async_copy(qsh.at[cidx.at[0]], qbuf.at[d],
                                  sg[d]).wait()

        def compute(d):
            def edge_body(k4, _):
                k0 = k4 * 4
                for dd in range(4):
                    k = k0 + dd
                    a0 = pbuf[d, k, pl.ds(0, LN)] + qbuf[d, k, pl.ds(0, LN)]
                    a1 = (pbuf[d, k, pl.ds(LN, LN)]
                          + qbuf[d, k, pl.ds(LN, LN)])
                    mbuf[d, k, pl.ds(0, LN)] = jnp.where(a0 > 0.0, 1.0, 0.0)
                    mbuf[d, k, pl.ds(LN, LN)] = jnp.where(a1 > 0.0, 1.0, 0.0)
                return 0
            lax.fori_loop(0, K // 4, edge_body, 0)

        def fire_scatter(j, d):
            pltpu.async_copy(mbuf.at[d], mrsh.at[ridx.at[j]], ss[d],
                             add=True)
            pltpu.async_copy(mbuf.at[d], mcsh.at[cidx.at[j]], ss[d],
                             add=True)

        def wait_scatter(d):
            pltpu.make_async_copy(mbuf.at[d], mrsh.at[ridx.at[0]],
                                  ss[d]).wait()
            pltpu.make_async_copy(mbuf.at[d], mcsh.at[cidx.at[0]],
                                  ss[d]).wait()

        def group_body(g, _):
            ge0 = pl.multiple_of(ei0 + g * (M * 2 * K), 8)
            pltpu.sync_copy(ei_hbm.at[pl.ds(ge0, M * 2 * K)], ibuf)
            lanes2 = 2 * lax.iota(jnp.int32, LN)

            def build_idx(j, _):
                base = j * (2 * K)
                for v in range(K // LN):
                    il = base + 2 * v * LN + lanes2
                    ridx[j, pl.ds(v * LN, LN)] = plsc.load_gather(ibuf, [il])
                    cidx[j, pl.ds(v * LN, LN)] = plsc.load_gather(
                        ibuf, [il + 1])
                return 0

            lax.fori_loop(0, M, build_idx, 0)
            fire_gather(0, 0)

            def pair_body(jh, _):
                j0 = jh * 2
                for d in (0, 1):
                    j = j0 + d
                    wait_gather(d)

                    @pl.when(j + 1 < M)
                    def _():
                        fire_gather(j + 1, 1 - d)

                    @pl.when(j >= 2)
                    def _():
                        wait_scatter(d)

                    compute(d)
                    fire_scatter(j, d)
                return 0

            lax.fori_loop(0, M // 2, pair_body, 0)
            dl = (M - 1) % 2
            if M % 2 == 1:
                wait_gather(dl)
                wait_scatter(dl)
                compute(dl)
                fire_scatter(M - 1, dl)
            wait_scatter(1 - dl)
            wait_scatter(dl)
            return 0

        lax.fori_loop(0, ngroups, group_body, 0)
        plsc.subcore_barrier()

        pltpu.sync_copy(mrsh.at[pl.ds(row0, rpt), :],
                        m_hbm.at[b, pl.ds(row0, rpt), pl.ds(pcol, HH)])
        pltpu.sync_copy(mcsh.at[pl.ds(row0, rpt), :],
                        m_hbm.at[b, pl.ds(row0, rpt), pl.ds(qcol, HH)])

        @pl.when(s == 0)
        def _():
            r0 = NS * rpt
            pltpu.sync_copy(mrsh.at[pl.ds(r0, nrem), :],
                            m_hbm.at[b, pl.ds(r0, nrem), pl.ds(pcol, HH)])
            pltpu.sync_copy(mcsh.at[pl.ds(r0, nrem), :],
                            m_hbm.at[b, pl.ds(r0, nrem), pl.ds(qcol, HH)])


def _zero_zb(zb):
    def zbody(i, _):
        zb[i, pl.ds(0, LN)] = jnp.zeros((LN,), jnp.float32)
        zb[i, pl.ds(LN, LN)] = jnp.zeros((LN,), jnp.float32)
        return 0
    lax.fori_loop(0, zb.shape[0], zbody, 0)


def _sc_entry(B, N, E, HH, pq_hbm, ei_hbm, m_hbm,
              psh, qsh, mrsh, mcsh, ibuf, ridx, cidx, pbuf, qbuf, mbuf, zb,
              sg0, sg1, ss0, ss1):
    _zero_zb(zb)
    _sc_body(B, N, E, HH, pq_hbm, ei_hbm, m_hbm,
             psh, qsh, mrsh, mcsh, ibuf, ridx, cidx, pbuf, qbuf, mbuf, zb,
             sg0, sg1, ss0, ss1)


def _make_sc(B, N, E, HH):
    mesh = plsc.VectorSubcoreMesh(core_axis_name="c", subcore_axis_name="s",
                                  num_cores=NC, num_subcores=NS)
    return pl.kernel(
        functools.partial(_sc_entry, B, N, E, HH),
        out_type=jax.ShapeDtypeStruct((B, N, 2 * NC * HH), jnp.float32),
        mesh=mesh,
        compiler_params=pltpu.CompilerParams(use_tc_tiling_on_sc=False,
                                             needs_layout_passes=False),
        scratch_types=[
            pltpu.VMEM_SHARED((N, HH), jnp.float32),
            pltpu.VMEM_SHARED((N, HH), jnp.float32),
            pltpu.VMEM_SHARED((N, HH), jnp.float32),
            pltpu.VMEM_SHARED((N, HH), jnp.float32),
            pltpu.VMEM((25 * 2 * K,), jnp.int32),
            pltpu.VMEM((25, K), jnp.int32),
            pltpu.VMEM((25, K), jnp.int32),
            pltpu.VMEM((2, K, HH), jnp.float32),
            pltpu.VMEM((2, K, HH), jnp.float32),
            pltpu.VMEM((2, K, HH), jnp.float32),
            pltpu.VMEM((78, HH), jnp.float32),
            pltpu.SemaphoreType.DMA,
            pltpu.SemaphoreType.DMA,
            pltpu.SemaphoreType.DMA,
            pltpu.SemaphoreType.DMA,
        ],
    )


def _fin_body(tv_ref, m_ref, u_ref, o_ref):
    o_ref[...] = (jnp.dot(m_ref[...], u_ref[...],
                          preferred_element_type=jnp.float32)
                  * tv_ref[pl.program_id(0)])


def _make_fin(B, N, C, H2):
    return pl.pallas_call(
        _fin_body,
        grid=(B,),
        in_specs=[
            pl.BlockSpec(memory_space=pltpu.SMEM),
            pl.BlockSpec((None, N, H2), lambda b: (b, 0, 0)),
            pl.BlockSpec((H2, C), lambda b: (0, 0)),
        ],
        out_specs=pl.BlockSpec((None, N, C), lambda b: (b, 0, 0)),
        out_shape=jax.ShapeDtypeStruct((B, N, C), jnp.float32),
    )


def kernel(x, t, x_initial, W1, b1, W2, b2, t_emb, alphas_cumprod):
    C = W1.shape[0] // 2
    H = W1.shape[1]
    B = x.shape[0]
    N = x.shape[1] // C
    E = x_initial.shape[1] // 2
    HH = H // NC
    nh = H // HH

    xr = x.reshape(B, N, C)
    w1s = W1.reshape(2, C, H)
    H2 = 2 * H

    w_cat = w1s.reshape(2, C, nh, HH).transpose(2, 0, 3, 1)
    w_cat = w_cat.reshape(H2, C).T
    cv = (b1[None, :] + t_emb[t]).reshape(B, nh, 1, HH)
    cv = jnp.concatenate([cv, jnp.zeros_like(cv)], axis=2)
    cv = cv.reshape(B, 1, H2)
    pq = _make_pq(B, N, C, H2)(xr, w_cat, cv)

    m = _make_sc(B, N, E, HH)(pq, x_initial.reshape(B * E * 2))

    w2 = W2[:, 0]
    u = jnp.transpose(w1s * w2[None, None, :], (0, 2, 1))
    u = u.reshape(2, nh, HH, C).transpose(1, 0, 2, 3)
    u = u.reshape(H2, C)
    tvec = 2.0 * (1.0 - alphas_cumprod)[t]

    grad = _make_fin(B, N, C, H2)(tvec, m, u)
    return grad.reshape(B, N * C)


# speedup vs baseline: 1.0635x; 1.0635x over previous
import functools

import jax
import jax.numpy as jnp
from jax import lax
from jax.experimental import pallas as pl
from jax.experimental.pallas import tpu as pltpu
from jax.experimental.pallas import tpu_sc as plsc

NC = 2
NS = 16
LN = 16
K = 80


def _pq_body(x_ref, w_ref, cv_ref, o_ref):
    o_ref[...] = (jnp.dot(x_ref[...], w_ref[...],
                          preferred_element_type=jnp.float32) + cv_ref[...])


def _make_pq(B, N, C, H2):
    return pl.pallas_call(
        _pq_body,
        grid=(B,),
        in_specs=[
            pl.BlockSpec((None, N, C), lambda b: (b, 0, 0)),
            pl.BlockSpec((C, H2), lambda b: (0, 0)),
            pl.BlockSpec((None, 1, H2), lambda b: (b, 0, 0)),
        ],
        out_specs=pl.BlockSpec((None, N, H2), lambda b: (b, 0, 0)),
        out_shape=jax.ShapeDtypeStruct((B, N, H2), jnp.float32),
    )


def _sc_body(B, N, E, HH, pq_hbm, ei_hbm, m_hbm,
             psh, qsh, mrsh, mcsh, ibuf, ridx, cidx, pbuf, qbuf, mbuf, zb,
             sg0, sg1, sg2, ss0, ss1, ss2):
    c = lax.axis_index("c")
    s = lax.axis_index("s")
    rpt = (N // NS) & ~7
    nrem = N - NS * rpt
    eper = E // NS
    nchunks = eper // K
    row0 = s * rpt
    sg = (sg0, sg1, sg2)
    ss = (ss0, ss1, ss2)
    M = ridx.shape[0]
    ngroups = nchunks // M

    pcol = pl.multiple_of(c * 2 * HH, 8)
    qcol = pl.multiple_of(c * 2 * HH + HH, 8)

    for b in range(B):
        pltpu.sync_copy(pq_hbm.at[b, pl.ds(row0, rpt), pl.ds(pcol, HH)],
                        psh.at[pl.ds(row0, rpt), :])
        pltpu.sync_copy(pq_hbm.at[b, pl.ds(row0, rpt), pl.ds(qcol, HH)],
                        qsh.at[pl.ds(row0, rpt), :])
        zr = zb.shape[0]
        for z in range(rpt // zr):
            pltpu.sync_copy(zb, mrsh.at[pl.ds(row0 + z * zr, zr), :])
            pltpu.sync_copy(zb, mcsh.at[pl.ds(row0 + z * zr, zr), :])

        @pl.when(s == 0)
        def _():
            r0 = NS * rpt
            pltpu.sync_copy(pq_hbm.at[b, pl.ds(r0, nrem), pl.ds(pcol, HH)],
                            psh.at[pl.ds(r0, nrem), :])
            pltpu.sync_copy(pq_hbm.at[b, pl.ds(r0, nrem), pl.ds(qcol, HH)],
                            qsh.at[pl.ds(r0, nrem), :])
            pltpu.sync_copy(zb.at[pl.ds(0, nrem), :],
                            mrsh.at[pl.ds(r0, nrem), :])
            pltpu.sync_copy(zb.at[pl.ds(0, nrem), :],
                            mcsh.at[pl.ds(r0, nrem), :])

        ei0 = (b * E + s * eper) * 2
        plsc.subcore_barrier()

        def fire_gather(j, d):
            pltpu.async_copy(psh.at[ridx.at[j]], pbuf.at[d], sg[d])
            pltpu.async_copy(qsh.at[cidx.at[j]], qbuf.at[d], sg[d])

        def wait_gather(d):
            pltpu.make_async_copy(psh.at[ridx.at[0]], pbuf.at[d],
                                  sg[d]).wait()
            pltpu.make_async_copy(qsh.at[cidx.at[0]], qbuf.at[d],
                                  sg[d]).wait()

        def compute(d):
            def edge_body(k4, _):
                k0 = k4 * 4
                for dd in range(4):
                    k = k0 + dd
                    a0 = pbuf[d, k, pl.ds(0, LN)] + qbuf[d, k, pl.ds(0, LN)]
                    a1 = (pbuf[d, k, pl.ds(LN, LN)]
                          + qbuf[d, k, pl.ds(LN, LN)])
                    mbuf[d, k, pl.ds(0, LN)] = jnp.where(a0 > 0.0, 1.0, 0.0)
                    mbuf[d, k, pl.ds(LN, LN)] = jnp.where(a1 > 0.0, 1.0, 0.0)
                return 0
            lax.fori_loop(0, K // 4, edge_body, 0)

        def fire_scatter(j, d):
            pltpu.async_copy(mbuf.at[d], mrsh.at[ridx.at[j]], ss[d],
                             add=True)
            pltpu.async_copy(mbuf.at[d], mcsh.at[cidx.at[j]], ss[d],
                             add=True)

        def wait_scatter(d):
            pltpu.make_async_copy(mbuf.at[d], mrsh.at[ridx.at[0]],
                                  ss[d]).wait()
            pltpu.make_async_copy(mbuf.at[d], mcsh.at[cidx.at[0]],
                                  ss[d]).wait()

        def group_body(g, _):
            ge0 = pl.multiple_of(ei0 + g * (M * 2 * K), 8)
            pltpu.sync_copy(ei_hbm.at[pl.ds(ge0, M * 2 * K)], ibuf)
            lanes2 = 2 * lax.iota(jnp.int32, LN)

            def build_idx(j, _):
                base = j * (2 * K)
                for v in range(K // LN):
                    il = base + 2 * v * LN + lanes2
                    ridx[j, pl.ds(v * LN, LN)] = plsc.load_gather(ibuf, [il])
                    cidx[j, pl.ds(v * LN, LN)] = plsc.load_gather(
                        ibuf, [il + 1])
                return 0

            lax.fori_loop(0, M, build_idx, 0)
            fire_gather(0, 0)
            fire_gather(1, 1)

            def tri_body(jh, _):
                j0 = jh * 3
                for d in (0, 1, 2):
                    j = j0 + d
                    wait_gather(d)

                    @pl.when(j + 2 < M)
                    def _():
                        fire_gather(j + 2, (d + 2) % 3)

                    @pl.when(j >= 3)
                    def _():
                        wait_scatter(d)

                    compute(d)
                    fire_scatter(j, d)
                return 0

            lax.fori_loop(0, M // 3, tri_body, 0)
            for j in range(3 * (M // 3), M):
                dl = j % 3
                wait_gather(dl)
                wait_scatter(dl)
                compute(dl)
                fire_scatter(j, dl)
            wait_scatter((M - 2) % 3)
            wait_scatter((M - 1) % 3)
            wait_scatter(M % 3)
            return 0

        lax.fori_loop(0, ngroups, group_body, 0)
        plsc.subcore_barrier()

        pltpu.sync_copy(mrsh.at[pl.ds(row0, rpt), :],
                        m_hbm.at[b, pl.ds(row0, rpt), pl.ds(pcol, HH)])
        pltpu.sync_copy(mcsh.at[pl.ds(row0, rpt), :],
                        m_hbm.at[b, pl.ds(row0, rpt), pl.ds(qcol, HH)])

        @pl.when(s == 0)
        def _():
            r0 = NS * rpt
            pltpu.sync_copy(mrsh.at[pl.ds(r0, nrem), :],
                            m_hbm.at[b, pl.ds(r0, nrem), pl.ds(pcol, HH)])
            pltpu.sync_copy(mcsh.at[pl.ds(r0, nrem), :],
                            m_hbm.at[b, pl.ds(r0, nrem), pl.ds(qcol, HH)])


def _zero_zb(zb):
    def zbody(i, _):
        zb[i, pl.ds(0, LN)] = jnp.zeros((LN,), jnp.float32)
        zb[i, pl.ds(LN, LN)] = jnp.zeros((LN,), jnp.float32)
        return 0
    lax.fori_loop(0, zb.shape[0], zbody, 0)


def _sc_entry(B, N, E, HH, pq_hbm, ei_hbm, m_hbm,
              psh, qsh, mrsh, mcsh, ibuf, ridx, cidx, pbuf, qbuf, mbuf, zb,
              sg0, sg1, sg2, ss0, ss1, ss2):
    _zero_zb(zb)
    _sc_body(B, N, E, HH, pq_hbm, ei_hbm, m_hbm,
             psh, qsh, mrsh, mcsh, ibuf, ridx, cidx, pbuf, qbuf, mbuf, zb,
             sg0, sg1, sg2, ss0, ss1, ss2)


def _make_sc(B, N, E, HH):
    mesh = plsc.VectorSubcoreMesh(core_axis_name="c", subcore_axis_name="s",
                                  num_cores=NC, num_subcores=NS)
    return pl.kernel(
        functools.partial(_sc_entry, B, N, E, HH),
        out_type=jax.ShapeDtypeStruct((B, N, 2 * NC * HH), jnp.float32),
        mesh=mesh,
        compiler_params=pltpu.CompilerParams(use_tc_tiling_on_sc=False,
                                             needs_layout_passes=False),
        scratch_types=[
            pltpu.VMEM_SHARED((N, HH), jnp.float32),
            pltpu.VMEM_SHARED((N, HH), jnp.float32),
            pltpu.VMEM_SHARED((N, HH), jnp.float32),
            pltpu.VMEM_SHARED((N, HH), jnp.float32),
            pltpu.VMEM((25 * 2 * K,), jnp.int32),
            pltpu.VMEM((25, K), jnp.int32),
            pltpu.VMEM((25, K), jnp.int32),
            pltpu.VMEM((3, K, HH), jnp.float32),
            pltpu.VMEM((3, K, HH), jnp.float32),
            pltpu.VMEM((3, K, HH), jnp.float32),
            pltpu.VMEM((78, HH), jnp.float32),
            pltpu.SemaphoreType.DMA,
            pltpu.SemaphoreType.DMA,
            pltpu.SemaphoreType.DMA,
            pltpu.SemaphoreType.DMA,
            pltpu.SemaphoreType.DMA,
            pltpu.SemaphoreType.DMA,
        ],
    )


def _fin_body(tv_ref, m_ref, u_ref, o_ref):
    o_ref[...] = (jnp.dot(m_ref[...], u_ref[...],
                          preferred_element_type=jnp.float32)
                  * tv_ref[pl.program_id(0)])


def _make_fin(B, N, C, H2):
    return pl.pallas_call(
        _fin_body,
        grid=(B,),
        in_specs=[
            pl.BlockSpec(memory_space=pltpu.SMEM),
            pl.BlockSpec((None, N, H2), lambda b: (b, 0, 0)),
            pl.BlockSpec((H2, C), lambda b: (0, 0)),
        ],
        out_specs=pl.BlockSpec((None, N, C), lambda b: (b, 0, 0)),
        out_shape=jax.ShapeDtypeStruct((B, N, C), jnp.float32),
    )


def kernel(x, t, x_initial, W1, b1, W2, b2, t_emb, alphas_cumprod):
    C = W1.shape[0] // 2
    H = W1.shape[1]
    B = x.shape[0]
    N = x.shape[1] // C
    E = x_initial.shape[1] // 2
    HH = H // NC
    nh = H // HH

    xr = x.reshape(B, N, C)
    w1s = W1.reshape(2, C, H)
    H2 = 2 * H

    w_cat = w1s.reshape(2, C, nh, HH).transpose(2, 0, 3, 1)
    w_cat = w_cat.reshape(H2, C).T
    cv = (b1[None, :] + t_emb[t]).reshape(B, nh, 1, HH)
    cv = jnp.concatenate([cv, jnp.zeros_like(cv)], axis=2)
    cv = cv.reshape(B, 1, H2)
    pq = _make_pq(B, N, C, H2)(xr, w_cat, cv)

    m = _make_sc(B, N, E, HH)(pq, x_initial.reshape(B * E * 2))

    w2 = W2[:, 0]
    u = jnp.transpose(w1s * w2[None, None, :], (0, 2, 1))
    u = u.reshape(2, nh, HH, C).transpose(1, 0, 2, 3)
    u = u.reshape(H2, C)
    tvec = 2.0 * (1.0 - alphas_cumprod)[t]

    grad = _make_fin(B, N, C, H2)(tvec, m, u)
    return grad.reshape(B, N * C)


# speedup vs baseline: 1.1116x; 1.0452x over previous
import functools

import jax
import jax.numpy as jnp
from jax import lax
from jax.experimental import pallas as pl
from jax.experimental.pallas import tpu as pltpu
from jax.experimental.pallas import tpu_sc as plsc

NC = 2
NS = 16
LN = 16
K = 80


def _pq_body(x_ref, w_ref, cv_ref, o_ref):
    o_ref[...] = (jnp.dot(x_ref[...], w_ref[...],
                          preferred_element_type=jnp.float32) + cv_ref[...])


def _make_pq(B, N, C, H2):
    return pl.pallas_call(
        _pq_body,
        grid=(B,),
        in_specs=[
            pl.BlockSpec((None, N, C), lambda b: (b, 0, 0)),
            pl.BlockSpec((C, H2), lambda b: (0, 0)),
            pl.BlockSpec((None, 1, H2), lambda b: (b, 0, 0)),
        ],
        out_specs=pl.BlockSpec((None, N, H2), lambda b: (b, 0, 0)),
        out_shape=jax.ShapeDtypeStruct((B, N, H2), jnp.float32),
    )


def _sc_body(B, N, E, HH, pq_hbm, ei_hbm, m_hbm,
             psh, qsh, mrsh, mcsh, ibuf, ridx, cidx, pbuf, qbuf, mbuf, zb,
             sg0, sg1, sg2, ss0, ss1, ss2):
    c = lax.axis_index("c")
    s = lax.axis_index("s")
    rpt = (N // NS) & ~7
    nrem = N - NS * rpt
    eper = E // NS
    nchunks = eper // K
    row0 = s * rpt
    sg = (sg0, sg1, sg2)
    ss = (ss0, ss1, ss2)
    M = ridx.shape[0]
    ngroups = nchunks // M

    pcol = pl.multiple_of(c * 2 * HH, 8)
    qcol = pl.multiple_of(c * 2 * HH + HH, 8)

    for b in range(B):
        zr = zb.shape[0]
        descs = [
            pltpu.async_copy(pq_hbm.at[b, pl.ds(row0, rpt), pl.ds(pcol, HH)],
                             psh.at[pl.ds(row0, rpt), :], sg0),
            pltpu.async_copy(pq_hbm.at[b, pl.ds(row0, rpt), pl.ds(qcol, HH)],
                             qsh.at[pl.ds(row0, rpt), :], sg1),
        ]
        for z in range(rpt // zr):
            descs.append(pltpu.async_copy(
                zb, mrsh.at[pl.ds(row0 + z * zr, zr), :], sg2))
            descs.append(pltpu.async_copy(
                zb, mcsh.at[pl.ds(row0 + z * zr, zr), :], sg2))

        @pl.when(s == 0)
        def _():
            r0 = NS * rpt
            rdescs = [
                pltpu.async_copy(
                    pq_hbm.at[b, pl.ds(r0, nrem), pl.ds(pcol, HH)],
                    psh.at[pl.ds(r0, nrem), :], ss0),
                pltpu.async_copy(
                    pq_hbm.at[b, pl.ds(r0, nrem), pl.ds(qcol, HH)],
                    qsh.at[pl.ds(r0, nrem), :], ss0),
                pltpu.async_copy(zb.at[pl.ds(0, nrem), :],
                                 mrsh.at[pl.ds(r0, nrem), :], ss0),
                pltpu.async_copy(zb.at[pl.ds(0, nrem), :],
                                 mcsh.at[pl.ds(r0, nrem), :], ss0),
            ]
            for d in rdescs:
                d.wait()

        for d in descs:
            d.wait()

        ei0 = (b * E + s * eper) * 2
        plsc.subcore_barrier()

        def fire_gather(j, d):
            pltpu.async_copy(psh.at[ridx.at[j]], pbuf.at[d], sg[d])
            pltpu.async_copy(qsh.at[cidx.at[j]], qbuf.at[d], sg[d])

        def wait_gather(d):
            pltpu.make_async_copy(psh.at[ridx.at[0]], pbuf.at[d],
                                  sg[d]).wait()
            pltpu.make_async_copy(qsh.at[cidx.at[0]], qbuf.at[d],
                                  sg[d]).wait()

        def compute(d):
            def edge_body(k4, _):
                k0 = k4 * 4
                for dd in range(4):
                    k = k0 + dd
                    a0 = pbuf[d, k, pl.ds(0, LN)] + qbuf[d, k, pl.ds(0, LN)]
                    a1 = (pbuf[d, k, pl.ds(LN, LN)]
                          + qbuf[d, k, pl.ds(LN, LN)])
                    mbuf[d, k, pl.ds(0, LN)] = jnp.where(a0 > 0.0, 1.0, 0.0)
                    mbuf[d, k, pl.ds(LN, LN)] = jnp.where(a1 > 0.0, 1.0, 0.0)
                return 0
            lax.fori_loop(0, K // 4, edge_body, 0)

        def fire_scatter(j, d):
            pltpu.async_copy(mbuf.at[d], mrsh.at[ridx.at[j]], ss[d],
                             add=True)
            pltpu.async_copy(mbuf.at[d], mcsh.at[cidx.at[j]], ss[d],
                             add=True)

        def wait_scatter(d):
            pltpu.make_async_copy(mbuf.at[d], mrsh.at[ridx.at[0]],
                                  ss[d]).wait()
            pltpu.make_async_copy(mbuf.at[d], mcsh.at[cidx.at[0]],
                                  ss[d]).wait()

        def group_body(g, _):
            ge0 = pl.multiple_of(ei0 + g * (M * 2 * K), 8)
            pltpu.sync_copy(ei_hbm.at[pl.ds(ge0, M * 2 * K)], ibuf)
            lanes2 = 2 * lax.iota(jnp.int32, LN)

            def build_idx(j, _):
                base = j * (2 * K)
                for v in range(K // LN):
                    il = base + 2 * v * LN + lanes2
                    ridx[j, pl.ds(v * LN, LN)] = plsc.load_gather(ibuf, [il])
                    cidx[j, pl.ds(v * LN, LN)] = plsc.load_gather(
                        ibuf, [il + 1])
                return 0

            lax.fori_loop(0, M, build_idx, 0)
            fire_gather(0, 0)
            fire_gather(1, 1)

            def tri_body(jh, _):
                j0 = jh * 3
                for d in (0, 1, 2):
                    j = j0 + d
                    wait_gather(d)

                    @pl.when(j + 2 < M)
                    def _():
                        fire_gather(j + 2, (d + 2) % 3)

                    @pl.when(j >= 3)
                    def _():
                        wait_scatter(d)

                    compute(d)
                    fire_scatter(j, d)
                return 0

            lax.fori_loop(0, M // 3, tri_body, 0)
            for j in range(3 * (M // 3), M):
                dl = j % 3
                wait_gather(dl)
                wait_scatter(dl)
                compute(dl)
                fire_scatter(j, dl)
            wait_scatter((M - 2) % 3)
            wait_scatter((M - 1) % 3)
            wait_scatter(M % 3)
            return 0

        lax.fori_loop(0, ngroups, group_body, 0)
        plsc.subcore_barrier()

        pltpu.sync_copy(mrsh.at[pl.ds(row0, rpt), :],
                        m_hbm.at[b, pl.ds(row0, rpt), pl.ds(pcol, HH)])
        pltpu.sync_copy(mcsh.at[pl.ds(row0, rpt), :],
                        m_hbm.at[b, pl.ds(row0, rpt), pl.ds(qcol, HH)])

        @pl.when(s == 0)
        def _():
            r0 = NS * rpt
            pltpu.sync_copy(mrsh.at[pl.ds(r0, nrem), :],
                            m_hbm.at[b, pl.ds(r0, nrem), pl.ds(pcol, HH)])
            pltpu.sync_copy(mcsh.at[pl.ds(r0, nrem), :],
                            m_hbm.at[b, pl.ds(r0, nrem), pl.ds(qcol, HH)])


def _zero_zb(zb):
    def zbody(i, _):
        zb[i, pl.ds(0, LN)] = jnp.zeros((LN,), jnp.float32)
        zb[i, pl.ds(LN, LN)] = jnp.zeros((LN,), jnp.float32)
        return 0
    lax.fori_loop(0, zb.shape[0], zbody, 0)


def _sc_entry(B, N, E, HH, pq_hbm, ei_hbm, m_hbm,
              psh, qsh, mrsh, mcsh, ibuf, ridx, cidx, pbuf, qbuf, mbuf, zb,
              sg0, sg1, sg2, ss0, ss1, ss2):
    _zero_zb(zb)
    _sc_body(B, N, E, HH, pq_hbm, ei_hbm, m_hbm,
             psh, qsh, mrsh, mcsh, ibuf, ridx, cidx, pbuf, qbuf, mbuf, zb,
             sg0, sg1, sg2, ss0, ss1, ss2)


def _make_sc(B, N, E, HH):
    mesh = plsc.VectorSubcoreMesh(core_axis_name="c", subcore_axis_name="s",
                                  num_cores=NC, num_subcores=NS)
    return pl.kernel(
        functools.partial(_sc_entry, B, N, E, HH),
        out_type=jax.ShapeDtypeStruct((B, N, 2 * NC * HH), jnp.float32),
        mesh=mesh,
        compiler_params=pltpu.CompilerParams(use_tc_tiling_on_sc=False,
                                             needs_layout_passes=False),
        scratch_types=[
            pltpu.VMEM_SHARED((N, HH), jnp.float32),
            pltpu.VMEM_SHARED((N, HH), jnp.float32),
            pltpu.VMEM_SHARED((N, HH), jnp.float32),
            pltpu.VMEM_SHARED((N, HH), jnp.float32),
            pltpu.VMEM((25 * 2 * K,), jnp.int32),
            pltpu.VMEM((25, K), jnp.int32),
            pltpu.VMEM((25, K), jnp.int32),
            pltpu.VMEM((3, K, HH), jnp.float32),
            pltpu.VMEM((3, K, HH), jnp.float32),
            pltpu.VMEM((3, K, HH), jnp.float32),
            pltpu.VMEM((78, HH), jnp.float32),
            pltpu.SemaphoreType.DMA,
            pltpu.SemaphoreType.DMA,
            pltpu.SemaphoreType.DMA,
            pltpu.SemaphoreType.DMA,
            pltpu.SemaphoreType.DMA,
            pltpu.SemaphoreType.DMA,
        ],
    )


def _fin_body(tv_ref, m_ref, u_ref, o_ref):
    o_ref[...] = (jnp.dot(m_ref[...], u_ref[...],
                          preferred_element_type=jnp.float32)
                  * tv_ref[pl.program_id(0)])


def _make_fin(B, N, C, H2):
    return pl.pallas_call(
        _fin_body,
        grid=(B,),
        in_specs=[
            pl.BlockSpec(memory_space=pltpu.SMEM),
            pl.BlockSpec((None, N, H2), lambda b: (b, 0, 0)),
            pl.BlockSpec((H2, C), lambda b: (0, 0)),
        ],
        out_specs=pl.BlockSpec((None, N, C), lambda b: (b, 0, 0)),
        out_shape=jax.ShapeDtypeStruct((B, N, C), jnp.float32),
    )


def kernel(x, t, x_initial, W1, b1, W2, b2, t_emb, alphas_cumprod):
    C = W1.shape[0] // 2
    H = W1.shape[1]
    B = x.shape[0]
    N = x.shape[1] // C
    E = x_initial.shape[1] // 2
    HH = H // NC
    nh = H // HH

    xr = x.reshape(B, N, C)
    w1s = W1.reshape(2, C, H)
    H2 = 2 * H

    w_cat = w1s.reshape(2, C, nh, HH).transpose(2, 0, 3, 1)
    w_cat = w_cat.reshape(H2, C).T
    cv = (b1[None, :] + t_emb[t]).reshape(B, nh, 1, HH)
    cv = jnp.concatenate([cv, jnp.zeros_like(cv)], axis=2)
    cv = cv.reshape(B, 1, H2)
    pq = _make_pq(B, N, C, H2)(xr, w_cat, cv)

    m = _make_sc(B, N, E, HH)(pq, x_initial.reshape(B * E * 2))

    w2 = W2[:, 0]
    u = jnp.transpose(w1s * w2[None, None, :], (0, 2, 1))
    u = u.reshape(2, nh, HH, C).transpose(1, 0, 2, 3)
    u = u.reshape(H2, C)
    tvec = 2.0 * (1.0 - alphas_cumprod)[t]

    grad = _make_fin(B, N, C, H2)(tvec, m, u)
    return grad.reshape(B, N * C)
